# baseline (speedup 1.0000x reference)
import functools
import math

import jax
import jax.numpy as jnp
from jax import lax
from jax.experimental import pallas as pl
from jax.experimental.pallas import tpu as pltpu
from jax.experimental.pallas import tpu_sc as plsc

EMB = 64
NC = 2
NS = 16
NW = NC * NS
LANES = 16
CHUNK = 128
SCALE = math.sqrt(EMB)


@functools.partial(jax.jit, static_argnames=())
def kernel(tokens, weight):
    batch, hist = tokens.shape
    total = batch * hist
    assert total % (NW * CHUNK) == 0
    n_chunks = total // (NW * CHUNK)
    b_per_w = n_chunks * CHUNK

    idx = tokens.reshape(NW, n_chunks, CHUNK).astype(jnp.int32)

    mesh = plsc.VectorSubcoreMesh(core_axis_name="c", subcore_axis_name="s")

    @functools.partial(
        pl.kernel,
        mesh=mesh,
        out_type=jax.ShapeDtypeStruct((total, EMB), jnp.float32),
        scratch_types=[
            pltpu.VMEM((n_chunks, CHUNK), jnp.int32),
            pltpu.VMEM((CHUNK, EMB), jnp.float32),
            pltpu.SemaphoreType.DMA,
        ],
        compiler_params=pltpu.CompilerParams(use_tc_tiling_on_sc=False),
    )
    def emb_kernel(idx_hbm, table_hbm, out_hbm, idx_v, rows_v, sem):
        wid = lax.axis_index("s") * NC + lax.axis_index("c")
        base = wid * b_per_w
        pltpu.sync_copy(idx_hbm.at[wid], idx_v)

        def chunk_body(g, carry):
            pltpu.async_copy(table_hbm.at[idx_v.at[g]], rows_v, sem).wait()

            def scale_row(r, c2):
                for c in range(EMB // LANES):
                    sl = pl.ds(c * LANES, LANES)
                    rows_v[r, sl] = rows_v[r, sl] * SCALE
                return c2

            lax.fori_loop(0, CHUNK, scale_row, 0)
            off = base + g * CHUNK
            pltpu.sync_copy(rows_v, out_hbm.at[pl.ds(off, CHUNK)])
            return carry

        lax.fori_loop(0, n_chunks, chunk_body, 0)

    out = emb_kernel(idx, weight)
    return out.reshape(batch, hist, EMB)


# speedup vs baseline: 1.2056x; 1.2056x over previous
import functools
import math

import jax
import jax.numpy as jnp
from jax import lax
from jax.experimental import pallas as pl
from jax.experimental.pallas import tpu as pltpu
from jax.experimental.pallas import tpu_sc as plsc

EMB = 64
NC = 2
NS = 16
NW = NC * NS
LANES = 16
CHUNK = 128
NBUF = 4
RPI = 8
SCALE = math.sqrt(EMB)


def kernel(tokens, weight):
    batch, hist = tokens.shape
    total = batch * hist
    assert total % (NW * CHUNK * NBUF) == 0
    n_chunks = total // (NW * CHUNK)
    n_rings = n_chunks // NBUF
    b_per_w = n_chunks * CHUNK

    idx = tokens.reshape(NW, n_chunks, CHUNK).astype(jnp.int32)

    mesh = plsc.VectorSubcoreMesh(core_axis_name="c", subcore_axis_name="s")

    @functools.partial(
        pl.kernel,
        mesh=mesh,
        out_type=jax.ShapeDtypeStruct((total, EMB), jnp.float32),
        scratch_types=[
            pltpu.VMEM((n_chunks, CHUNK), jnp.int32),
            pltpu.VMEM((NBUF, CHUNK, EMB), jnp.float32),
            pltpu.VMEM((NBUF, CHUNK, EMB), jnp.float32),
            pltpu.SemaphoreType.DMA((NBUF,)),
            pltpu.SemaphoreType.DMA((NBUF,)),
        ],
        compiler_params=pltpu.CompilerParams(use_tc_tiling_on_sc=False),
    )
    def emb_kernel(idx_hbm, table_hbm, out_hbm, idx_v, rows_in, rows_out,
                   gsem, osem):
        wid = lax.axis_index("s") * NC + lax.axis_index("c")
        base = wid * b_per_w
        pltpu.sync_copy(idx_hbm.at[wid], idx_v)

        def gather(g, b):
            pltpu.async_copy(table_hbm.at[idx_v.at[g]], rows_in.at[b],
                             gsem.at[b])

        def wait_gather(g, b):
            pltpu.make_async_copy(table_hbm.at[idx_v.at[g]], rows_in.at[b],
                                  gsem.at[b]).wait()

        def out_copy(g, b):
            off = base + g * CHUNK
            pltpu.async_copy(rows_out.at[b], out_hbm.at[pl.ds(off, CHUNK)],
                             osem.at[b])

        def wait_out(g, b):
            off = base + g * CHUNK
            pltpu.make_async_copy(rows_out.at[b], out_hbm.at[pl.ds(off, CHUNK)],
                                  osem.at[b]).wait()

        def scale(b):
            src = rows_in.at[b]
            dst = rows_out.at[b]

            def blk(i, c2):
                for rr in range(RPI):
                    r = i * RPI + rr
                    for c in range(EMB // LANES):
                        sl = pl.ds(c * LANES, LANES)
                        dst[r, sl] = src[r, sl] * SCALE
                return c2

            lax.fori_loop(0, CHUNK // RPI, blk, 0)

        for b in range(NBUF):
            gather(b, b)

        for b in range(NBUF):
            wait_gather(b, b)
            scale(b)
            gather(b + NBUF, b)
            out_copy(b, b)

        def ring(r, carry):
            for b in range(NBUF):
                g = r * NBUF + b
                wait_gather(g, b)
                wait_out(g - NBUF, b)
                scale(b)
                gather(g + NBUF, b)
                out_copy(g, b)
            return carry

        lax.fori_loop(1, n_rings - 1, ring, 0)

        for b in range(NBUF):
            g = (n_rings - 1) * NBUF + b
            wait_gather(g, b)
            wait_out(g - NBUF, b)
            scale(b)
            out_copy(g, b)
        for b in range(NBUF):
            wait_out((n_rings - 1) * NBUF + b, b)

    out = emb_kernel(idx, weight)
    return out.reshape(batch, hist, EMB)
